# baseline (speedup 1.0000x reference)
import functools

import jax
import jax.numpy as jnp
from jax import lax
from jax.experimental import pallas as pl
from jax.experimental.pallas import tpu as pltpu
from jax.experimental.pallas import tpu_sc as plsc

N = 10000
NPAD = 10240
E = 320000
NTILES = 16
ROWS_PT = NPAD // NTILES
NB = 160
BE = 128
EPAD = NTILES * NB * BE
FS = 16
NSLAB = 128 // FS


def _sc_body(h8, row3, col3, w3, u1_out, u2_out, hs_out,
             t_row, t_col, t_w, t_g0, t_g1, t_s0, t_s1, t_buf, t_zero,
             t_vec, sm_deg, sm_acc, sem_b0, sem_b1):
    c = lax.axis_index("c")
    s = lax.axis_index("s")
    n0 = s * ROWS_PT

    pltpu.sync_copy(row3.at[s], t_row)
    pltpu.sync_copy(col3.at[s], t_col)
    pltpu.sync_copy(w3.at[s], t_w)

    cbase = c * ((NSLAB // 2) * NPAD)

    def _bias_col(b, _):
        for k in range(BE // 16):
            t_col[b, pl.ds(16 * k, 16)] = t_col[b, pl.ds(16 * k, 16)] + cbase
        return 0
    lax.fori_loop(0, NB, _bias_col, 0)

    def _zero_zbuf(n, _):
        t_zero[n, :] = jnp.zeros((FS,), jnp.float32)
        return 0
    lax.fori_loop(0, ROWS_PT, _zero_zbuf, 0)

    def _zero_vec(i, _):
        t_vec[pl.ds(16 * i, 16)] = jnp.zeros((16,), jnp.float32)
        return 0
    lax.fori_loop(0, ROWS_PT // 16, _zero_vec, 0)

    pltpu.sync_copy(t_vec, sm_deg.at[pl.ds(n0, ROWS_PT)])
    pltpu.sync_copy(t_zero, sm_acc.at[pl.ds(n0, ROWS_PT)])
    plsc.subcore_barrier()

    def _deg_batch(b, _):
        pltpu.sync_copy(t_w.at[b], sm_deg.at[t_row.at[b]], add=True)
        return 0
    lax.fori_loop(0, NB, _deg_batch, 0)
    plsc.subcore_barrier()

    pltpu.sync_copy(sm_deg.at[pl.ds(n0, ROWS_PT)], t_vec)

    def _dinv_chunk(i, _):
        d = t_vec[pl.ds(16 * i, 16)]
        di = lax.bitcast_convert_type(d, jnp.int32)
        yi = jnp.int32(0x5F3759DF) - lax.shift_right_logical(di, 1)
        y = lax.bitcast_convert_type(yi, jnp.float32)
        y = y * (1.5 - 0.5 * d * y * y)
        y = y * (1.5 - 0.5 * d * y * y)
        y = y * (1.5 - 0.5 * d * y * y)
        t_vec[pl.ds(16 * i, 16)] = jnp.where(d > 0.0, y, 0.0)
        return 0
    lax.fori_loop(0, ROWS_PT // 16, _dinv_chunk, 0)

    def _scale_rows(j, _):
        dv16 = t_vec[pl.ds(16 * j, 16)]
        base = 16 * j
        for l in range(16):
            t_buf[base + l, :] = t_buf[base + l, :] * dv16[l]
        return 0

    for q in range(NSLAB // 2):
        qq = (NSLAB // 2) * c + q
        pltpu.sync_copy(h8.at[qq, pl.ds(n0, ROWS_PT)], t_buf)
        lax.fori_loop(0, ROWS_PT // 16, _scale_rows, 0)
        pltpu.sync_copy(t_buf, hs_out.at[pl.ds(qq * NPAD + n0, ROWS_PT)])
    plsc.subcore_barrier()

    bufs = ((t_g0, t_s0, sem_b0), (t_g1, t_s1, sem_b1))

    def _sweep():
        pltpu.async_copy(hs_out.at[t_col.at[0]], t_g0, sem_b0)
        pltpu.async_copy(hs_out.at[t_col.at[1]], t_g1, sem_b1)

        def _pair(g, _):
            for p in range(2):
                tg, ts, sb = bufs[p]
                b = 2 * g + p
                pltpu.make_async_copy(hs_out.at[pl.ds(0, BE)], tg, sb).wait()

                def _scale_edge(j, _):
                    w16 = t_w[b, pl.ds(16 * j, 16)]
                    base = 16 * j
                    for l in range(16):
                        ts[base + l, :] = tg[base + l, :] * w16[l]
                    return 0
                lax.fori_loop(0, BE // 16, _scale_edge, 0)
                bn = jnp.minimum(b + 2, NB - 1)
                pltpu.async_copy(hs_out.at[t_col.at[bn]], tg, sb)
                pltpu.sync_copy(ts, sm_acc.at[t_row.at[b]], add=True)
            return 0
        lax.fori_loop(0, NB // 2, _pair, 0)
        for p in range(2):
            tg, ts, sb = bufs[p]
            pltpu.make_async_copy(hs_out.at[pl.ds(0, BE)], tg, sb).wait()
        plsc.subcore_barrier()

    def _drain_acc():
        pltpu.sync_copy(sm_acc.at[pl.ds(n0, ROWS_PT)], t_buf)
        pltpu.sync_copy(t_zero, sm_acc.at[pl.ds(n0, ROWS_PT)])
        lax.fori_loop(0, ROWS_PT // 16, _scale_rows, 0)

    for q in range(NSLAB // 2):
        qq = (NSLAB // 2) * c + q
        slab0 = qq * NPAD + n0
        _sweep()
        _drain_acc()
        pltpu.sync_copy(t_buf, u1_out.at[pl.ds(slab0, ROWS_PT)])
        lax.fori_loop(0, ROWS_PT // 16, _scale_rows, 0)
        pltpu.sync_copy(t_buf, hs_out.at[pl.ds(slab0, ROWS_PT)])
        plsc.subcore_barrier()
        _sweep()
        _drain_acc()
        pltpu.sync_copy(t_buf, u2_out.at[pl.ds(slab0, ROWS_PT)])
        plsc.subcore_barrier()
        if q < NSLAB // 2 - 1:
            def _rebias_col(b, _):
                for k in range(BE // 16):
                    t_col[b, pl.ds(16 * k, 16)] = (
                        t_col[b, pl.ds(16 * k, 16)] + NPAD)
                return 0
            lax.fori_loop(0, NB, _rebias_col, 0)


_sc_cheb = functools.partial(
    pl.kernel,
    out_type=(
        jax.ShapeDtypeStruct((NSLAB * NPAD, FS), jnp.float32),
        jax.ShapeDtypeStruct((NSLAB * NPAD, FS), jnp.float32),
        jax.ShapeDtypeStruct((NSLAB * NPAD, FS), jnp.float32),
    ),
    mesh=plsc.VectorSubcoreMesh(core_axis_name="c", subcore_axis_name="s"),
    compiler_params=pltpu.CompilerParams(use_tc_tiling_on_sc=False),
    scratch_types=[
        pltpu.VMEM((NB, BE), jnp.int32),
        pltpu.VMEM((NB, BE), jnp.int32),
        pltpu.VMEM((NB, BE), jnp.float32),
        pltpu.VMEM((BE, FS), jnp.float32),
        pltpu.VMEM((BE, FS), jnp.float32),
        pltpu.VMEM((BE, FS), jnp.float32),
        pltpu.VMEM((BE, FS), jnp.float32),
        pltpu.VMEM((ROWS_PT, FS), jnp.float32),
        pltpu.VMEM((ROWS_PT, FS), jnp.float32),
        pltpu.VMEM((ROWS_PT,), jnp.float32),
        pltpu.VMEM_SHARED((NPAD,), jnp.float32),
        pltpu.VMEM_SHARED((NPAD, FS), jnp.float32),
        pltpu.SemaphoreType.DMA,
        pltpu.SemaphoreType.DMA,
    ],
)(_sc_body)


BN = 2000


def _tc_body(x, h, cc, *rest):
    us = rest[:2 * NSLAB]
    wx, wh = rest[2 * NSLAB], rest[2 * NSLAB + 1]
    ws = rest[2 * NSLAB + 2:4 * NSLAB + 2]
    bias = rest[4 * NSLAB + 2]
    h_out, c_out = rest[4 * NSLAB + 3], rest[4 * NSLAB + 4]
    acc = jnp.dot(x[...], wx[...], preferred_element_type=jnp.float32)
    acc += jnp.dot(h[...], wh[...], preferred_element_type=jnp.float32)
    for u, w in zip(us, ws):
        acc += jnp.dot(u[...], w[...], preferred_element_type=jnp.float32)
    acc += bias[0:1, :]
    ig = jax.nn.sigmoid(acc[:, 0:128])
    fg = jax.nn.sigmoid(acc[:, 128:256])
    tg = jnp.tanh(acc[:, 256:384])
    og = jax.nn.sigmoid(acc[:, 384:512])
    c_new = fg * cc[...] + ig * tg
    h_out[...] = og * jnp.tanh(c_new)
    c_out[...] = c_new


def _tc_gates(x, h, c, u1s, u2s, wx, wh, w1s, w2s, bias):
    grid = (N // BN,)
    row_spec = pl.BlockSpec((BN, 128), lambda i: (i, 0))
    slab_spec = pl.BlockSpec((BN, FS), lambda i: (i, 0))
    w128 = pl.BlockSpec((128, 512), lambda i: (0, 0))
    w16 = pl.BlockSpec((FS, 512), lambda i: (0, 0))
    bspec = pl.BlockSpec((8, 512), lambda i: (0, 0))
    return pl.pallas_call(
        _tc_body,
        grid=grid,
        in_specs=[row_spec] * 3 + [slab_spec] * (2 * NSLAB)
                 + [w128, w128] + [w16] * (2 * NSLAB) + [bspec],
        out_specs=[row_spec, row_spec],
        out_shape=[jax.ShapeDtypeStruct((N, 128), jnp.float32),
                   jax.ShapeDtypeStruct((N, 128), jnp.float32)],
    )(x, h, c, *u1s, *u2s, wx, wh, *w1s, *w2s, bias)


def kernel(X, edge_index, edge_weight, H, C,
           W_i, b_i, Theta_i, bconv_i,
           W_f, b_f, Theta_f, bconv_f,
           W_c, b_c, Theta_c, bconv_c,
           W_o, b_o, Theta_o, bconv_o):
    pad_idx = (jnp.arange(EPAD - E, dtype=jnp.int32) % (NPAD - N)) + N
    row3 = jnp.concatenate(
        [edge_index[0].astype(jnp.int32), pad_idx]).reshape(NTILES, NB, BE)
    col3 = jnp.concatenate(
        [edge_index[1].astype(jnp.int32), pad_idx]).reshape(NTILES, NB, BE)
    w3 = jnp.concatenate(
        [edge_weight.astype(jnp.float32),
         jnp.zeros((EPAD - E,), jnp.float32)]).reshape(NTILES, NB, BE)
    h_pad = jnp.pad(H, ((0, NPAD - N), (0, 0)))
    h8 = h_pad.reshape(NPAD, NSLAB, FS).transpose(1, 0, 2)

    u1f, u2f, _hs = _sc_cheb(h8, row3, col3, w3)
    u1s = [u1f[qq * NPAD:(qq + 1) * NPAD] for qq in range(NSLAB)]
    u2s = [u2f[qq * NPAD:(qq + 1) * NPAD] for qq in range(NSLAB)]

    gates = [(W_i, b_i, Theta_i, bconv_i), (W_f, b_f, Theta_f, bconv_f),
             (W_c, b_c, Theta_c, bconv_c), (W_o, b_o, Theta_o, bconv_o)]
    wx = jnp.concatenate([g[0] for g in gates], axis=1)
    wh = jnp.concatenate([g[2][0] - g[2][2] for g in gates], axis=1)
    w1 = jnp.concatenate([-g[2][1] for g in gates], axis=1)
    w2 = jnp.concatenate([2.0 * g[2][2] for g in gates], axis=1)
    w1s = [w1[qq * FS:(qq + 1) * FS] for qq in range(NSLAB)]
    w2s = [w2[qq * FS:(qq + 1) * FS] for qq in range(NSLAB)]
    bias = jnp.concatenate(
        [g[1].reshape(-1) + g[3] for g in gates]).reshape(1, 512)
    bias = jnp.broadcast_to(bias, (8, 512))

    h_new, c_new = _tc_gates(X, H, C, u1s, u2s, wx, wh, w1s, w2s, bias)
    return (h_new, c_new)


# speedup vs baseline: 1.2641x; 1.2641x over previous
import functools

import jax
import jax.numpy as jnp
from jax import lax
from jax.experimental import pallas as pl
from jax.experimental.pallas import tpu as pltpu
from jax.experimental.pallas import tpu_sc as plsc

N = 10000
NPAD = 10240
E = 320000
NTILES = 16
ROWS_PT = NPAD // NTILES
NB = 160
BE = 128
EPAD = NTILES * NB * BE
FS = 16
NSLAB = 128 // FS


def _sc_body(h8, row3, col3, w3, u1_out, u2_out, hs_out,
             t_row, t_col, t_w, t_g0, t_g1, t_s0, t_s1, t_buf, t_zero,
             t_vec, sm_deg, sm_acc, sem_b0, sem_b1):
    c = lax.axis_index("c")
    s = lax.axis_index("s")
    n0 = s * ROWS_PT

    pltpu.sync_copy(row3.at[s], t_row)
    pltpu.sync_copy(col3.at[s], t_col)
    pltpu.sync_copy(w3.at[s], t_w)

    cbase = c * ((NSLAB // 2) * NPAD)

    def _bias_col(b, _):
        for k in range(BE // 16):
            t_col[b, pl.ds(16 * k, 16)] = t_col[b, pl.ds(16 * k, 16)] + cbase
        return 0
    lax.fori_loop(0, NB, _bias_col, 0)

    def _zero_zbuf(n, _):
        t_zero[n, :] = jnp.zeros((FS,), jnp.float32)
        return 0
    lax.fori_loop(0, ROWS_PT, _zero_zbuf, 0)

    def _zero_vec(i, _):
        t_vec[pl.ds(16 * i, 16)] = jnp.zeros((16,), jnp.float32)
        return 0
    lax.fori_loop(0, ROWS_PT // 16, _zero_vec, 0)

    pltpu.sync_copy(t_vec, sm_deg.at[pl.ds(n0, ROWS_PT)])
    pltpu.sync_copy(t_zero, sm_acc.at[pl.ds(n0, ROWS_PT)])
    plsc.subcore_barrier()

    def _deg_batch(b, _):
        pltpu.sync_copy(t_w.at[b], sm_deg.at[t_row.at[b]], add=True)
        return 0
    lax.fori_loop(0, NB, _deg_batch, 0)
    plsc.subcore_barrier()

    pltpu.sync_copy(sm_deg.at[pl.ds(n0, ROWS_PT)], t_vec)

    def _dinv_chunk(i, _):
        d = t_vec[pl.ds(16 * i, 16)]
        di = lax.bitcast_convert_type(d, jnp.int32)
        yi = jnp.int32(0x5F3759DF) - lax.shift_right_logical(di, 1)
        y = lax.bitcast_convert_type(yi, jnp.float32)
        y = y * (1.5 - 0.5 * d * y * y)
        y = y * (1.5 - 0.5 * d * y * y)
        y = y * (1.5 - 0.5 * d * y * y)
        t_vec[pl.ds(16 * i, 16)] = jnp.where(d > 0.0, y, 0.0)
        return 0
    lax.fori_loop(0, ROWS_PT // 16, _dinv_chunk, 0)

    def _scale_rows(j, _):
        dv16 = t_vec[pl.ds(16 * j, 16)]
        base = 16 * j
        for l in range(16):
            t_buf[base + l, :] = t_buf[base + l, :] * dv16[l]
        return 0

    for q in range(NSLAB // 2):
        qq = (NSLAB // 2) * c + q
        pltpu.sync_copy(
            h8.at[pl.ds(n0, ROWS_PT), pl.ds(qq * FS, FS)], t_buf)
        lax.fori_loop(0, ROWS_PT // 16, _scale_rows, 0)
        pltpu.sync_copy(t_buf, hs_out.at[pl.ds(qq * NPAD + n0, ROWS_PT)])
    plsc.subcore_barrier()

    bufs = ((t_g0, t_s0, sem_b0), (t_g1, t_s1, sem_b1))

    def _sweep():
        pltpu.async_copy(hs_out.at[t_col.at[0]], t_g0, sem_b0)
        pltpu.async_copy(hs_out.at[t_col.at[1]], t_g1, sem_b1)

        def _pair(g, _):
            for p in range(2):
                tg, ts, sb = bufs[p]
                b = 2 * g + p
                pltpu.make_async_copy(hs_out.at[pl.ds(0, BE)], tg, sb).wait()

                def _scale_edge(j, _):
                    w16 = t_w[b, pl.ds(16 * j, 16)]
                    base = 16 * j
                    for l in range(16):
                        ts[base + l, :] = tg[base + l, :] * w16[l]
                    return 0
                lax.fori_loop(0, BE // 16, _scale_edge, 0)
                bn = jnp.minimum(b + 2, NB - 1)
                pltpu.async_copy(hs_out.at[t_col.at[bn]], tg, sb)
                pltpu.sync_copy(ts, sm_acc.at[t_row.at[b]], add=True)
            return 0
        lax.fori_loop(0, NB // 2, _pair, 0)
        for p in range(2):
            tg, ts, sb = bufs[p]
            pltpu.make_async_copy(hs_out.at[pl.ds(0, BE)], tg, sb).wait()
        plsc.subcore_barrier()

    def _drain_acc():
        pltpu.sync_copy(sm_acc.at[pl.ds(n0, ROWS_PT)], t_buf)
        pltpu.sync_copy(t_zero, sm_acc.at[pl.ds(n0, ROWS_PT)])
        lax.fori_loop(0, ROWS_PT // 16, _scale_rows, 0)

    for q in range(NSLAB // 2):
        qq = (NSLAB // 2) * c + q
        slab0 = qq * NPAD + n0
        _sweep()
        _drain_acc()
        pltpu.sync_copy(
            t_buf, u1_out.at[pl.ds(n0, ROWS_PT), pl.ds(qq * FS, FS)])
        lax.fori_loop(0, ROWS_PT // 16, _scale_rows, 0)
        pltpu.sync_copy(t_buf, hs_out.at[pl.ds(slab0, ROWS_PT)])
        plsc.subcore_barrier()
        _sweep()
        _drain_acc()
        pltpu.sync_copy(
            t_buf, u2_out.at[pl.ds(n0, ROWS_PT), pl.ds(qq * FS, FS)])
        plsc.subcore_barrier()
        if q < NSLAB // 2 - 1:
            def _rebias_col(b, _):
                for k in range(BE // 16):
                    t_col[b, pl.ds(16 * k, 16)] = (
                        t_col[b, pl.ds(16 * k, 16)] + NPAD)
                return 0
            lax.fori_loop(0, NB, _rebias_col, 0)


_sc_cheb = functools.partial(
    pl.kernel,
    out_type=(
        jax.ShapeDtypeStruct((NPAD, 128), jnp.float32),
        jax.ShapeDtypeStruct((NPAD, 128), jnp.float32),
        jax.ShapeDtypeStruct((NSLAB * NPAD, FS), jnp.float32),
    ),
    mesh=plsc.VectorSubcoreMesh(core_axis_name="c", subcore_axis_name="s"),
    compiler_params=pltpu.CompilerParams(use_tc_tiling_on_sc=False),
    scratch_types=[
        pltpu.VMEM((NB, BE), jnp.int32),
        pltpu.VMEM((NB, BE), jnp.int32),
        pltpu.VMEM((NB, BE), jnp.float32),
        pltpu.VMEM((BE, FS), jnp.float32),
        pltpu.VMEM((BE, FS), jnp.float32),
        pltpu.VMEM((BE, FS), jnp.float32),
        pltpu.VMEM((BE, FS), jnp.float32),
        pltpu.VMEM((ROWS_PT, FS), jnp.float32),
        pltpu.VMEM((ROWS_PT, FS), jnp.float32),
        pltpu.VMEM((ROWS_PT,), jnp.float32),
        pltpu.VMEM_SHARED((NPAD,), jnp.float32),
        pltpu.VMEM_SHARED((NPAD, FS), jnp.float32),
        pltpu.SemaphoreType.DMA,
        pltpu.SemaphoreType.DMA,
    ],
)(_sc_body)


BN = 2000


def _tc_body(x, h, cc, u1, u2, wx, wh, w1, w2, bias, h_out, c_out):
    acc = jnp.dot(x[...], wx[...], preferred_element_type=jnp.float32)
    acc += jnp.dot(h[...], wh[...], preferred_element_type=jnp.float32)
    acc += jnp.dot(u1[...], w1[...], preferred_element_type=jnp.float32)
    acc += jnp.dot(u2[...], w2[...], preferred_element_type=jnp.float32)
    acc += bias[0:1, :]
    ig = jax.nn.sigmoid(acc[:, 0:128])
    fg = jax.nn.sigmoid(acc[:, 128:256])
    tg = jnp.tanh(acc[:, 256:384])
    og = jax.nn.sigmoid(acc[:, 384:512])
    c_new = fg * cc[...] + ig * tg
    h_out[...] = og * jnp.tanh(c_new)
    c_out[...] = c_new


def _tc_gates(x, h, c, u1, u2, wx, wh, w1, w2, bias):
    grid = (N // BN,)
    row_spec = pl.BlockSpec((BN, 128), lambda i: (i, 0))
    w128 = pl.BlockSpec((128, 512), lambda i: (0, 0))
    bspec = pl.BlockSpec((8, 512), lambda i: (0, 0))
    return pl.pallas_call(
        _tc_body,
        grid=grid,
        in_specs=[row_spec] * 5 + [w128] * 4 + [bspec],
        out_specs=[row_spec, row_spec],
        out_shape=[jax.ShapeDtypeStruct((N, 128), jnp.float32),
                   jax.ShapeDtypeStruct((N, 128), jnp.float32)],
    )(x, h, c, u1, u2, wx, wh, w1, w2, bias)


def kernel(X, edge_index, edge_weight, H, C,
           W_i, b_i, Theta_i, bconv_i,
           W_f, b_f, Theta_f, bconv_f,
           W_c, b_c, Theta_c, bconv_c,
           W_o, b_o, Theta_o, bconv_o):
    pad_idx = (jnp.arange(EPAD - E, dtype=jnp.int32) % (NPAD - N)) + N
    row3 = jnp.concatenate(
        [edge_index[0].astype(jnp.int32), pad_idx]).reshape(NTILES, NB, BE)
    col3 = jnp.concatenate(
        [edge_index[1].astype(jnp.int32), pad_idx]).reshape(NTILES, NB, BE)
    w3 = jnp.concatenate(
        [edge_weight.astype(jnp.float32),
         jnp.zeros((EPAD - E,), jnp.float32)]).reshape(NTILES, NB, BE)
    h_pad = jnp.pad(H, ((0, NPAD - N), (0, 0)))

    u1, u2, _hs = _sc_cheb(h_pad, row3, col3, w3)

    gates = [(W_i, b_i, Theta_i, bconv_i), (W_f, b_f, Theta_f, bconv_f),
             (W_c, b_c, Theta_c, bconv_c), (W_o, b_o, Theta_o, bconv_o)]
    wx = jnp.concatenate([g[0] for g in gates], axis=1)
    wh = jnp.concatenate([g[2][0] - g[2][2] for g in gates], axis=1)
    w1 = jnp.concatenate([-g[2][1] for g in gates], axis=1)
    w2 = jnp.concatenate([2.0 * g[2][2] for g in gates], axis=1)
    bias = jnp.concatenate(
        [g[1].reshape(-1) + g[3] for g in gates]).reshape(1, 512)
    bias = jnp.broadcast_to(bias, (8, 512))

    h_new, c_new = _tc_gates(X, H, C, u1, u2, wx, wh, w1, w2, bias)
    return (h_new, c_new)


# speedup vs baseline: 1.2693x; 1.0041x over previous
import functools

import jax
import jax.numpy as jnp
from jax import lax
from jax.experimental import pallas as pl
from jax.experimental.pallas import tpu as pltpu
from jax.experimental.pallas import tpu_sc as plsc

N = 10000
NPAD = 10240
E = 320000
NTILES = 16
ROWS_PT = NPAD // NTILES
NB = 160
BE = 128
EPAD = NTILES * NB * BE
FS = 16
NSLAB = 128 // FS


def _sc_body(h8, row3, col3, w3, u1_out, u2_out, hs_out,
             t_row, t_col, t_w, t_g0, t_g1, t_s0, t_s1, t_buf, t_buf2,
             t_zero, t_vec, sm_deg, sm_acc, sem_b0, sem_b1):
    c = lax.axis_index("c")
    s = lax.axis_index("s")
    n0 = s * ROWS_PT

    pltpu.sync_copy(row3.at[s], t_row)
    pltpu.sync_copy(col3.at[s], t_col)
    pltpu.sync_copy(w3.at[s], t_w)

    cbase = c * ((NSLAB // 2) * NPAD)

    def _bias_col(b, _):
        for k in range(BE // 16):
            t_col[b, pl.ds(16 * k, 16)] = t_col[b, pl.ds(16 * k, 16)] + cbase
        return 0
    lax.fori_loop(0, NB, _bias_col, 0)

    def _zero_zbuf(n, _):
        t_zero[n, :] = jnp.zeros((FS,), jnp.float32)
        return 0
    lax.fori_loop(0, ROWS_PT, _zero_zbuf, 0)

    def _zero_vec(i, _):
        t_vec[pl.ds(16 * i, 16)] = jnp.zeros((16,), jnp.float32)
        return 0
    lax.fori_loop(0, ROWS_PT // 16, _zero_vec, 0)

    pltpu.sync_copy(t_vec, sm_deg.at[pl.ds(n0, ROWS_PT)])
    pltpu.sync_copy(t_zero, sm_acc.at[pl.ds(n0, ROWS_PT)])
    plsc.subcore_barrier()

    def _deg_batch(b, _):
        pltpu.sync_copy(t_w.at[b], sm_deg.at[t_row.at[b]], add=True)
        return 0
    lax.fori_loop(0, NB, _deg_batch, 0)
    plsc.subcore_barrier()

    pltpu.sync_copy(sm_deg.at[pl.ds(n0, ROWS_PT)], t_vec)

    def _dinv_chunk(i, _):
        d = t_vec[pl.ds(16 * i, 16)]
        di = lax.bitcast_convert_type(d, jnp.int32)
        yi = jnp.int32(0x5F3759DF) - lax.shift_right_logical(di, 1)
        y = lax.bitcast_convert_type(yi, jnp.float32)
        y = y * (1.5 - 0.5 * d * y * y)
        y = y * (1.5 - 0.5 * d * y * y)
        y = y * (1.5 - 0.5 * d * y * y)
        t_vec[pl.ds(16 * i, 16)] = jnp.where(d > 0.0, y, 0.0)
        return 0
    lax.fori_loop(0, ROWS_PT // 16, _dinv_chunk, 0)

    def _scale_rows(j, _):
        dv16 = t_vec[pl.ds(16 * j, 16)]
        base = 16 * j
        for l in range(16):
            t_buf[base + l, :] = t_buf[base + l, :] * dv16[l]
        return 0

    for q in range(NSLAB // 2):
        qq = (NSLAB // 2) * c + q
        pltpu.sync_copy(
            h8.at[pl.ds(n0, ROWS_PT), pl.ds(qq * FS, FS)], t_buf)
        lax.fori_loop(0, ROWS_PT // 16, _scale_rows, 0)
        pltpu.sync_copy(t_buf, hs_out.at[pl.ds(qq * NPAD + n0, ROWS_PT)])
    plsc.subcore_barrier()

    bufs = ((t_g0, t_s0, sem_b0), (t_g1, t_s1, sem_b1))

    def _sweep():
        pltpu.async_copy(hs_out.at[t_col.at[0]], t_g0, sem_b0)
        pltpu.async_copy(hs_out.at[t_col.at[1]], t_g1, sem_b1)

        def _pair(g, _):
            for p in range(2):
                tg, ts, sb = bufs[p]
                b = 2 * g + p
                pltpu.make_async_copy(hs_out.at[pl.ds(0, BE)], tg, sb).wait()

                def _scale_edge(j, _):
                    w16 = t_w[b, pl.ds(16 * j, 16)]
                    base = 16 * j
                    for l in range(16):
                        ts[base + l, :] = tg[base + l, :] * w16[l]
                    return 0
                lax.fori_loop(0, BE // 16, _scale_edge, 0)
                bn = jnp.minimum(b + 2, NB - 1)
                pltpu.async_copy(hs_out.at[t_col.at[bn]], tg, sb)
                pltpu.sync_copy(ts, sm_acc.at[t_row.at[b]], add=True)
            return 0
        lax.fori_loop(0, NB // 2, _pair, 0)
        for p in range(2):
            tg, ts, sb = bufs[p]
            pltpu.make_async_copy(hs_out.at[pl.ds(0, BE)], tg, sb).wait()
        plsc.subcore_barrier()

    def _drain_acc():
        pltpu.sync_copy(sm_acc.at[pl.ds(n0, ROWS_PT)], t_buf)
        pltpu.sync_copy(t_zero, sm_acc.at[pl.ds(n0, ROWS_PT)])
        lax.fori_loop(0, ROWS_PT // 16, _scale_rows, 0)

    def _drain_acc2(j, _):
        dv16 = t_vec[pl.ds(16 * j, 16)]
        base = 16 * j
        for l in range(16):
            u = t_buf[base + l, :] * dv16[l]
            t_buf2[base + l, :] = u
            t_buf[base + l, :] = u * dv16[l]
        return 0

    for q in range(NSLAB // 2):
        qq = (NSLAB // 2) * c + q
        slab0 = qq * NPAD + n0
        _sweep()
        pltpu.sync_copy(sm_acc.at[pl.ds(n0, ROWS_PT)], t_buf)
        pltpu.async_copy(t_zero, sm_acc.at[pl.ds(n0, ROWS_PT)], sem_b0)
        lax.fori_loop(0, ROWS_PT // 16, _drain_acc2, 0)
        pltpu.sync_copy(
            t_buf2, u1_out.at[pl.ds(n0, ROWS_PT), pl.ds(qq * FS, FS)])
        pltpu.sync_copy(t_buf, hs_out.at[pl.ds(slab0, ROWS_PT)])
        pltpu.make_async_copy(
            t_zero, sm_acc.at[pl.ds(n0, ROWS_PT)], sem_b0).wait()
        plsc.subcore_barrier()
        _sweep()
        _drain_acc()
        pltpu.sync_copy(
            t_buf, u2_out.at[pl.ds(n0, ROWS_PT), pl.ds(qq * FS, FS)])
        plsc.subcore_barrier()

        if q < NSLAB // 2 - 1:
            def _rebias_col(b, _):
                for k in range(BE // 16):
                    t_col[b, pl.ds(16 * k, 16)] = (
                        t_col[b, pl.ds(16 * k, 16)] + NPAD)
                return 0
            lax.fori_loop(0, NB, _rebias_col, 0)


_sc_cheb = functools.partial(
    pl.kernel,
    out_type=(
        jax.ShapeDtypeStruct((NPAD, 128), jnp.float32),
        jax.ShapeDtypeStruct((NPAD, 128), jnp.float32),
        jax.ShapeDtypeStruct((NSLAB * NPAD, FS), jnp.float32),
    ),
    mesh=plsc.VectorSubcoreMesh(core_axis_name="c", subcore_axis_name="s"),
    compiler_params=pltpu.CompilerParams(use_tc_tiling_on_sc=False),
    scratch_types=[
        pltpu.VMEM((NB, BE), jnp.int32),
        pltpu.VMEM((NB, BE), jnp.int32),
        pltpu.VMEM((NB, BE), jnp.float32),
        pltpu.VMEM((BE, FS), jnp.float32),
        pltpu.VMEM((BE, FS), jnp.float32),
        pltpu.VMEM((BE, FS), jnp.float32),
        pltpu.VMEM((BE, FS), jnp.float32),
        pltpu.VMEM((ROWS_PT, FS), jnp.float32),
        pltpu.VMEM((ROWS_PT, FS), jnp.float32),
        pltpu.VMEM((ROWS_PT, FS), jnp.float32),
        pltpu.VMEM((ROWS_PT,), jnp.float32),
        pltpu.VMEM_SHARED((NPAD,), jnp.float32),
        pltpu.VMEM_SHARED((NPAD, FS), jnp.float32),
        pltpu.SemaphoreType.DMA,
        pltpu.SemaphoreType.DMA,
    ],
)(_sc_body)


BN = 2000


def _tc_body(x, h, cc, u1, u2, wx, wh, w1, w2, bias, h_out, c_out):
    acc = jnp.dot(x[...], wx[...], preferred_element_type=jnp.float32)
    acc += jnp.dot(h[...], wh[...], preferred_element_type=jnp.float32)
    acc += jnp.dot(u1[...], w1[...], preferred_element_type=jnp.float32)
    acc += jnp.dot(u2[...], w2[...], preferred_element_type=jnp.float32)
    acc += bias[0:1, :]
    ig = jax.nn.sigmoid(acc[:, 0:128])
    fg = jax.nn.sigmoid(acc[:, 128:256])
    tg = jnp.tanh(acc[:, 256:384])
    og = jax.nn.sigmoid(acc[:, 384:512])
    c_new = fg * cc[...] + ig * tg
    h_out[...] = og * jnp.tanh(c_new)
    c_out[...] = c_new


def _tc_gates(x, h, c, u1, u2, wx, wh, w1, w2, bias):
    grid = (N // BN,)
    row_spec = pl.BlockSpec((BN, 128), lambda i: (i, 0))
    w128 = pl.BlockSpec((128, 512), lambda i: (0, 0))
    bspec = pl.BlockSpec((8, 512), lambda i: (0, 0))
    return pl.pallas_call(
        _tc_body,
        grid=grid,
        in_specs=[row_spec] * 5 + [w128] * 4 + [bspec],
        out_specs=[row_spec, row_spec],
        out_shape=[jax.ShapeDtypeStruct((N, 128), jnp.float32),
                   jax.ShapeDtypeStruct((N, 128), jnp.float32)],
    )(x, h, c, u1, u2, wx, wh, w1, w2, bias)


def kernel(X, edge_index, edge_weight, H, C,
           W_i, b_i, Theta_i, bconv_i,
           W_f, b_f, Theta_f, bconv_f,
           W_c, b_c, Theta_c, bconv_c,
           W_o, b_o, Theta_o, bconv_o):
    pad_idx = (jnp.arange(EPAD - E, dtype=jnp.int32) % (NPAD - N)) + N
    row3 = jnp.concatenate(
        [edge_index[0].astype(jnp.int32), pad_idx]).reshape(NTILES, NB, BE)
    col3 = jnp.concatenate(
        [edge_index[1].astype(jnp.int32), pad_idx]).reshape(NTILES, NB, BE)
    w3 = jnp.concatenate(
        [edge_weight.astype(jnp.float32),
         jnp.zeros((EPAD - E,), jnp.float32)]).reshape(NTILES, NB, BE)
    h_pad = jnp.pad(H, ((0, NPAD - N), (0, 0)))

    u1, u2, _hs = _sc_cheb(h_pad, row3, col3, w3)

    gates = [(W_i, b_i, Theta_i, bconv_i), (W_f, b_f, Theta_f, bconv_f),
             (W_c, b_c, Theta_c, bconv_c), (W_o, b_o, Theta_o, bconv_o)]
    wx = jnp.concatenate([g[0] for g in gates], axis=1)
    wh = jnp.concatenate([g[2][0] - g[2][2] for g in gates], axis=1)
    w1 = jnp.concatenate([-g[2][1] for g in gates], axis=1)
    w2 = jnp.concatenate([2.0 * g[2][2] for g in gates], axis=1)
    bias = jnp.concatenate(
        [g[1].reshape(-1) + g[3] for g in gates]).reshape(1, 512)
    bias = jnp.broadcast_to(bias, (8, 512))

    h_new, c_new = _tc_gates(X, H, C, u1, u2, wx, wh, w1, w2, bias)
    return (h_new, c_new)
